# Initial kernel scaffold; baseline (speedup 1.0000x reference)
import functools

import jax
import jax.numpy as jnp
from jax import lax
from jax.experimental import pallas as pl
from jax.experimental.pallas import tpu as pltpu
from jax.experimental.pallas import tpu_sc as plsc

B = 16384
D = 4
N_SPARSE = 26
N_DENSE = 13
N_SEQ = 5
SEQ_LEN = 20
N_SINGLE = N_SPARSE + N_DENSE
NCOL = N_SINGLE + N_SEQ * SEQ_LEN
NC, NS = 2, 16
NW = NC * NS
BPW = B // NW

_mesh = plsc.VectorSubcoreMesh(core_axis_name="c", subcore_axis_name="s")


def _body(*refs):
    idx_refs = refs[:N_SINGLE]
    w_refs = refs[N_SINGLE:2 * N_SINGLE]
    seq_refs = refs[2 * N_SINGLE:2 * N_SINGLE + N_SEQ]
    seqw_refs = refs[2 * N_SINGLE + N_SEQ:2 * N_SINGLE + 2 * N_SEQ]
    out = refs[2 * N_SINGLE + 2 * N_SEQ]
    idx_v, rows_v, idxq_v, rowsq_v, sem = refs[2 * N_SINGLE + 2 * N_SEQ + 1:]

    wid = lax.axis_index("c") * NS + lax.axis_index("s")
    base = wid * BPW

    for t in range(N_SINGLE):
        pltpu.sync_copy(idx_refs[t].at[pl.ds(base, BPW)], idx_v)
        pltpu.async_copy(w_refs[t].at[idx_v], rows_v, sem).wait()
        pltpu.sync_copy(rows_v, out.at[pl.ds(base, BPW), t])

    for s in range(N_SEQ):
        off = N_SINGLE + s * SEQ_LEN
        pltpu.sync_copy(seq_refs[s].at[pl.ds(base * SEQ_LEN, BPW * SEQ_LEN)],
                        idxq_v)
        pltpu.async_copy(seqw_refs[s].at[idxq_v], rowsq_v, sem).wait()
        pltpu.sync_copy(rowsq_v, out.at[pl.ds(base, BPW), pl.ds(off, SEQ_LEN)])


_call = functools.partial(
    pl.kernel,
    out_type=jax.ShapeDtypeStruct((B, NCOL, D), jnp.float32),
    mesh=_mesh,
    scratch_types=[
        pltpu.VMEM((BPW,), jnp.int32),
        pltpu.VMEM((BPW, D), jnp.float32),
        pltpu.VMEM((BPW * SEQ_LEN,), jnp.int32),
        pltpu.VMEM((BPW * SEQ_LEN, D), jnp.float32),
        pltpu.SemaphoreType.DMA,
    ],
)(_body)


def kernel(sparse_0, W_sparse_0, sparse_1, W_sparse_1, sparse_2, W_sparse_2, sparse_3, W_sparse_3, sparse_4, W_sparse_4, sparse_5, W_sparse_5, sparse_6, W_sparse_6, sparse_7, W_sparse_7, sparse_8, W_sparse_8, sparse_9, W_sparse_9, sparse_10, W_sparse_10, sparse_11, W_sparse_11, sparse_12, W_sparse_12, sparse_13, W_sparse_13, sparse_14, W_sparse_14, sparse_15, W_sparse_15, sparse_16, W_sparse_16, sparse_17, W_sparse_17, sparse_18, W_sparse_18, sparse_19, W_sparse_19, sparse_20, W_sparse_20, sparse_21, W_sparse_21, sparse_22, W_sparse_22, sparse_23, W_sparse_23, sparse_24, W_sparse_24, sparse_25, W_sparse_25, dense_0, W_dense_0, dense_1, W_dense_1, dense_2, W_dense_2, dense_3, W_dense_3, dense_4, W_dense_4, dense_5, W_dense_5, dense_6, W_dense_6, dense_7, W_dense_7, dense_8, W_dense_8, dense_9, W_dense_9, dense_10, W_dense_10, dense_11, W_dense_11, dense_12, W_dense_12, register_game_seq, W_register_game_seq, active_game_seq, W_active_game_seq, pay_game_seq, W_pay_game_seq, onlinetime_seq, W_onlinetime_seq, payment_seq, W_payment_seq):
    kw = dict(locals())
    idxs = [kw[f"sparse_{i}"] for i in range(N_SPARSE)]
    idxs += [kw[f"dense_{i}"] for i in range(N_DENSE)]
    ws = [kw[f"W_sparse_{i}"] for i in range(N_SPARSE)]
    ws += [kw[f"W_dense_{i}"] for i in range(N_DENSE)]
    seq_names = ["register_game_seq", "active_game_seq", "pay_game_seq",
                 "onlinetime_seq", "payment_seq"]
    seqs = [kw[n].reshape(-1) for n in seq_names]
    seqws = [kw["W_" + n] for n in seq_names]
    return _call(*idxs, *ws, *seqs, *seqws)


# speedup vs baseline: 3.7272x; 3.7272x over previous
import functools

import jax
import jax.numpy as jnp
from jax import lax
from jax.experimental import pallas as pl
from jax.experimental.pallas import tpu as pltpu
from jax.experimental.pallas import tpu_sc as plsc

B = 16384
D = 4
N_SPARSE = 26
N_DENSE = 13
N_SEQ = 5
SEQ_LEN = 20
N_SINGLE = N_SPARSE + N_DENSE
NCOL = N_SINGLE + N_SEQ * SEQ_LEN
ROWF = NCOL * D
NC, NS = 2, 16
NW = NC * NS
BPW = B // NW
C = 64
NCHUNK = BPW // C

DENSE_ROWS = 50
SEQ_ROWS = [500, 500, 500, 5, 5]
SMALL_ROWS = N_DENSE * DENSE_ROWS + sum(SEQ_ROWS)
SPROWS0 = SMALL_ROWS
VAL_ROWS = SPROWS0 + N_SPARSE * C

_mesh = plsc.VectorSubcoreMesh(core_axis_name="c", subcore_axis_name="s")


def _body(*refs):
    w_refs = refs[:N_SPARSE]
    small_hbm = refs[N_SPARSE]
    idx_refs = refs[N_SPARSE + 1:N_SPARSE + 1 + N_SINGLE]
    seq_refs = refs[N_SPARSE + 1 + N_SINGLE:
                    N_SPARSE + 1 + N_SINGLE + N_SEQ]
    out = refs[N_SPARSE + 1 + N_SINGLE + N_SEQ]
    tbl_v, chunk_v, idx_v, idxh_v, semA, semB = \
        refs[N_SPARSE + 2 + N_SINGLE + N_SEQ:]

    wid = lax.axis_index("c") * NS + lax.axis_index("s")
    base = wid * BPW

    pltpu.sync_copy(small_hbm, tbl_v.at[pl.ds(0, SMALL_ROWS), :])

    lane = lax.iota(jnp.int32, 16)
    lane4 = lane >> 2
    dvec = lane & 3
    GPC = C * D // 16

    def chunk_body(ci, _):
        cb = base + ci * C

        cps = []
        for t in range(N_SINGLE):
            cps.append(pltpu.async_copy(
                idx_refs[t].at[pl.ds(cb, C)],
                idx_v.at[pl.ds(t * C, C)], semA))
        for s in range(N_SEQ):
            cps.append(pltpu.async_copy(
                seq_refs[s].at[pl.ds(cb * SEQ_LEN, C * SEQ_LEN)],
                idx_v.at[pl.ds((N_SINGLE + s * SEQ_LEN) * C, C * SEQ_LEN)],
                semA))
        for cp in cps:
            cp.wait()

        def h_body(g, _):
            v = idx_v[pl.ds(g * 16, 16)]
            idxh_v[pl.ds(g * 16, 16)] = v >> 1
            return ()
        lax.fori_loop(0, N_SPARSE * C // 16, h_body, ())

        gps = []
        for t in range(N_SPARSE):
            gps.append(pltpu.async_copy(
                w_refs[t].at[idxh_v.at[pl.ds(t * C, C)]],
                tbl_v.at[pl.ds(SPROWS0 + t * C, C), :], semB))

        def s_body(s, _):
            A = jnp.where(s < 3, 650 + s * 500, 2150 + (s - 3) * 5)
            iof0 = (N_SINGLE + s * SEQ_LEN) * C
            oof0 = (N_SINGLE + s * SEQ_LEN) * D

            def k_body(k, _):
                def g_body(g, _):
                    cvec = g * 4 + lane4
                    iv = plsc.load_gather(idx_v, [iof0 + k + cvec * SEQ_LEN])
                    col8 = ((iv & 1) << 2) + dvec
                    val = plsc.load_gather(tbl_v, [A + (iv >> 1), col8])
                    plsc.store_scatter(
                        chunk_v, [cvec * ROWF + oof0 + k * 4 + dvec], val)
                    return ()
                lax.fori_loop(0, GPC, g_body, ())
                return ()
            lax.fori_loop(0, SEQ_LEN, k_body, ())
            return ()
        lax.fori_loop(0, N_SEQ, s_body, ())

        for gp in gps:
            gp.wait()

        def t_body(t, _):
            flag = t < N_SPARSE
            rb = jnp.where(flag, SPROWS0 + t * C, (t - N_SPARSE) * DENSE_ROWS)

            def g_body(g, _):
                cvec = g * 4 + lane4
                iv = plsc.load_gather(idx_v, [t * C + cvec])
                col8 = ((iv & 1) << 2) + dvec
                rowv = jnp.where(flag, rb + cvec, rb + (iv >> 1))
                val = plsc.load_gather(tbl_v, [rowv, col8])
                plsc.store_scatter(chunk_v, [cvec * ROWF + t * 4 + dvec], val)
                return ()
            lax.fori_loop(0, GPC, g_body, ())
            return ()
        lax.fori_loop(0, N_SINGLE, t_body, ())

        pltpu.sync_copy(chunk_v, out.at[pl.ds(cb * ROWF, C * ROWF)])
        return ()

    lax.fori_loop(0, NCHUNK, chunk_body, ())


_call = functools.partial(
    pl.kernel,
    out_type=jax.ShapeDtypeStruct((B * ROWF,), jnp.float32),
    mesh=_mesh,
    compiler_params=pltpu.CompilerParams(use_tc_tiling_on_sc=False,
                                         needs_layout_passes=False),
    scratch_types=[
        pltpu.VMEM((VAL_ROWS, 8), jnp.float32),
        pltpu.VMEM((C * ROWF,), jnp.float32),
        pltpu.VMEM((NCOL * C,), jnp.int32),
        pltpu.VMEM((N_SPARSE * C,), jnp.int32),
        pltpu.SemaphoreType.DMA,
        pltpu.SemaphoreType.DMA,
    ],
)(_body)


def kernel(sparse_0, W_sparse_0, sparse_1, W_sparse_1, sparse_2, W_sparse_2, sparse_3, W_sparse_3, sparse_4, W_sparse_4, sparse_5, W_sparse_5, sparse_6, W_sparse_6, sparse_7, W_sparse_7, sparse_8, W_sparse_8, sparse_9, W_sparse_9, sparse_10, W_sparse_10, sparse_11, W_sparse_11, sparse_12, W_sparse_12, sparse_13, W_sparse_13, sparse_14, W_sparse_14, sparse_15, W_sparse_15, sparse_16, W_sparse_16, sparse_17, W_sparse_17, sparse_18, W_sparse_18, sparse_19, W_sparse_19, sparse_20, W_sparse_20, sparse_21, W_sparse_21, sparse_22, W_sparse_22, sparse_23, W_sparse_23, sparse_24, W_sparse_24, sparse_25, W_sparse_25, dense_0, W_dense_0, dense_1, W_dense_1, dense_2, W_dense_2, dense_3, W_dense_3, dense_4, W_dense_4, dense_5, W_dense_5, dense_6, W_dense_6, dense_7, W_dense_7, dense_8, W_dense_8, dense_9, W_dense_9, dense_10, W_dense_10, dense_11, W_dense_11, dense_12, W_dense_12, register_game_seq, W_register_game_seq, active_game_seq, W_active_game_seq, pay_game_seq, W_pay_game_seq, onlinetime_seq, W_onlinetime_seq, payment_seq, W_payment_seq):
    kw = dict(locals())
    seq_names = ["register_game_seq", "active_game_seq", "pay_game_seq",
                 "onlinetime_seq", "payment_seq"]
    ws = [kw[f"W_sparse_{i}"].reshape(-1, 8) for i in range(N_SPARSE)]
    small = jnp.concatenate(
        [kw[f"W_dense_{i}"].reshape(-1, 8) for i in range(N_DENSE)]
        + [kw["W_" + n].reshape(-1, 8) for n in seq_names], axis=0)
    idxs = [kw[f"sparse_{i}"] for i in range(N_SPARSE)]
    idxs += [kw[f"dense_{i}"] for i in range(N_DENSE)]
    seqs = [kw[n].reshape(-1) for n in seq_names]
    flat = _call(*ws, small, *idxs, *seqs)
    return flat.reshape(B, NCOL, D)


# speedup vs baseline: 6.8106x; 1.8273x over previous
import functools

import jax
import jax.numpy as jnp
from jax import lax
from jax.experimental import pallas as pl
from jax.experimental.pallas import tpu as pltpu
from jax.experimental.pallas import tpu_sc as plsc

B = 16384
D = 4
N_SPARSE = 26
N_DENSE = 13
N_SEQ = 5
SEQ_LEN = 20
N_SINGLE = N_SPARSE + N_DENSE
NCOL = N_SINGLE + N_SEQ * SEQ_LEN
ROWF = NCOL * D
NC, NS = 2, 16
NW = NC * NS
BPW = B // NW
C = 32
NCHUNK = BPW // C
GPC = C * D // 16

DENSE_ROWS = 64
SEQ_ROWS = [512, 512, 512, 64, 64]
SEQ_OFF = [N_DENSE * DENSE_ROWS + sum(SEQ_ROWS[:i]) for i in range(N_SEQ)]
SMALL_ROWS = N_DENSE * DENSE_ROWS + sum(SEQ_ROWS)
SPROWS0 = SMALL_ROWS
VAL_ROWS = SPROWS0 + N_SPARSE * 4 * C

_mesh = plsc.VectorSubcoreMesh(core_axis_name="c", subcore_axis_name="s")


def _body(*refs):
    w_refs = refs[:N_SPARSE]
    small_hbm = refs[N_SPARSE]
    idxm_hbm = refs[N_SPARSE + 1]
    out = refs[N_SPARSE + 2]
    (tbl_v, chunk0_v, chunk1_v, idx_v, glist_v,
     semA, semB, semO0, semO1) = refs[N_SPARSE + 3:]

    wid = lax.axis_index("c") * NS + lax.axis_index("s")
    base = wid * BPW

    pltpu.sync_copy(small_hbm, tbl_v.at[pl.ds(0, SMALL_ROWS), :])

    lane = lax.iota(jnp.int32, 16)
    lane4 = lane >> 2
    dvec = lane & 3

    def do_chunk(ci2, p, chunk_v, semO):
        ci = ci2 * 2 + p
        cb = base + ci * C

        @pl.when(ci2 > 0)
        def _():
            pltpu.make_async_copy(
                chunk_v, out.at[pl.ds(0, C * ROWF)], semO).wait()

        pltpu.sync_copy(idxm_hbm.at[:, pl.ds(cb, C)], idx_v)

        GQ = C // 4
        def gl_body(q, _):
            tvec = jnp.full((16,), 0, jnp.int32) + (q >> 3)
            cvec = (q & 7) * 4 + lane4
            iv = plsc.load_gather(idx_v, [tvec, cvec])
            r = ((iv >> 7) << 6) + (dvec << 4) + ((iv >> 3) & 15)
            glist_v[pl.ds(q * 16, 16)] = r
            return ()
        lax.fori_loop(0, N_SPARSE * GQ, gl_body, ())

        gps = []
        for t in range(N_SPARSE):
            gps.append(pltpu.async_copy(
                w_refs[t].at[glist_v.at[pl.ds(t * 4 * C, 4 * C)]],
                tbl_v.at[pl.ds(SPROWS0 + t * 4 * C, 4 * C), :], semB))

        for s in range(N_SEQ):
            A = SEQ_OFF[s]

            def k_body(k, _, s=s, A=A):
                jrow = jnp.full((16,), 0, jnp.int32) + (N_SINGLE + s * SEQ_LEN + k)
                oof = (N_SINGLE + s * SEQ_LEN + k) * D
                for g in range(GPC):
                    cvec = g * 4 + lane4
                    iv = plsc.load_gather(idx_v, [jrow, cvec])
                    rowv = A + ((iv >> 7) << 6) + (dvec << 4) + ((iv >> 3) & 15)
                    val = plsc.load_gather(tbl_v, [rowv, iv & 7])
                    plsc.store_scatter(
                        chunk_v, [cvec * ROWF + oof + dvec], val)
                return ()
            lax.fori_loop(0, SEQ_LEN, k_body, ())

        for gp in gps:
            gp.wait()

        def t_body(t, _):
            flag = t < N_SPARSE
            rb = jnp.where(flag, SPROWS0 + t * 4 * C,
                           (t - N_SPARSE) * DENSE_ROWS)
            jrow = jnp.full((16,), 0, jnp.int32) + t
            for g in range(GPC):
                cvec = g * 4 + lane4
                iv = plsc.load_gather(idx_v, [jrow, cvec])
                rowv = jnp.where(
                    flag, rb + cvec * 4 + dvec,
                    rb + ((iv >> 7) << 6) + (dvec << 4) + ((iv >> 3) & 15))
                val = plsc.load_gather(tbl_v, [rowv, iv & 7])
                plsc.store_scatter(chunk_v, [cvec * ROWF + t * 4 + dvec], val)
            return ()
        lax.fori_loop(0, N_SINGLE, t_body, ())

        pltpu.async_copy(chunk_v, out.at[pl.ds(cb * ROWF, C * ROWF)], semO)

    def chunk_body(ci2, _):
        do_chunk(ci2, 0, chunk0_v, semO0)
        do_chunk(ci2, 1, chunk1_v, semO1)
        return ()

    lax.fori_loop(0, NCHUNK // 2, chunk_body, ())

    pltpu.make_async_copy(chunk0_v, out.at[pl.ds(0, C * ROWF)], semO0).wait()
    pltpu.make_async_copy(chunk1_v, out.at[pl.ds(0, C * ROWF)], semO1).wait()


_call = functools.partial(
    pl.kernel,
    out_type=jax.ShapeDtypeStruct((B * ROWF,), jnp.float32),
    mesh=_mesh,
    compiler_params=pltpu.CompilerParams(use_tc_tiling_on_sc=False,
                                         needs_layout_passes=False),
    scratch_types=[
        pltpu.VMEM((VAL_ROWS, 8), jnp.float32),
        pltpu.VMEM((C * ROWF,), jnp.float32),
        pltpu.VMEM((C * ROWF,), jnp.float32),
        pltpu.VMEM((NCOL, C), jnp.int32),
        pltpu.VMEM((N_SPARSE * 4 * C,), jnp.int32),
        pltpu.SemaphoreType.DMA,
        pltpu.SemaphoreType.DMA,
        pltpu.SemaphoreType.DMA,
        pltpu.SemaphoreType.DMA,
    ],
)(_body)


def kernel(sparse_0, W_sparse_0, sparse_1, W_sparse_1, sparse_2, W_sparse_2, sparse_3, W_sparse_3, sparse_4, W_sparse_4, sparse_5, W_sparse_5, sparse_6, W_sparse_6, sparse_7, W_sparse_7, sparse_8, W_sparse_8, sparse_9, W_sparse_9, sparse_10, W_sparse_10, sparse_11, W_sparse_11, sparse_12, W_sparse_12, sparse_13, W_sparse_13, sparse_14, W_sparse_14, sparse_15, W_sparse_15, sparse_16, W_sparse_16, sparse_17, W_sparse_17, sparse_18, W_sparse_18, sparse_19, W_sparse_19, sparse_20, W_sparse_20, sparse_21, W_sparse_21, sparse_22, W_sparse_22, sparse_23, W_sparse_23, sparse_24, W_sparse_24, sparse_25, W_sparse_25, dense_0, W_dense_0, dense_1, W_dense_1, dense_2, W_dense_2, dense_3, W_dense_3, dense_4, W_dense_4, dense_5, W_dense_5, dense_6, W_dense_6, dense_7, W_dense_7, dense_8, W_dense_8, dense_9, W_dense_9, dense_10, W_dense_10, dense_11, W_dense_11, dense_12, W_dense_12, register_game_seq, W_register_game_seq, active_game_seq, W_active_game_seq, pay_game_seq, W_pay_game_seq, onlinetime_seq, W_onlinetime_seq, payment_seq, W_payment_seq):
    kw = dict(locals())
    seq_names = ["register_game_seq", "active_game_seq", "pay_game_seq",
                 "onlinetime_seq", "payment_seq"]
    def _view8(w):
        v = w.shape[0]
        vp = -(-v // 128) * 128
        wp = jnp.pad(w, ((0, vp - v), (0, 0)))
        return wp.reshape(vp // 128, 128, 4).transpose(0, 2, 1).reshape(-1, 8)

    ws = [_view8(kw[f"W_sparse_{i}"]) for i in range(N_SPARSE)]
    small = jnp.concatenate(
        [_view8(kw[f"W_dense_{i}"]) for i in range(N_DENSE)]
        + [_view8(kw["W_" + n]) for n in seq_names], axis=0)
    idxm = jnp.concatenate(
        [jnp.stack([kw[f"sparse_{i}"] for i in range(N_SPARSE)]
                   + [kw[f"dense_{i}"] for i in range(N_DENSE)], axis=0)]
        + [kw[n].T for n in seq_names], axis=0)
    flat = _call(*ws, small, idxm)
    return flat.reshape(B, NCOL, D)


# speedup vs baseline: 25.8316x; 3.7928x over previous
import functools

import jax
import jax.numpy as jnp
from jax import lax
from jax.experimental import pallas as pl
from jax.experimental.pallas import tpu as pltpu
from jax.experimental.pallas import tpu_sc as plsc

B = 16384
D = 4
N_SPARSE = 26
N_DENSE = 13
N_SEQ = 5
SEQ_LEN = 20
N_SINGLE = N_SPARSE + N_DENSE
NCOL = N_SINGLE + N_SEQ * SEQ_LEN
ROWF = NCOL * D
NC, NS = 2, 16
NW = NC * NS
BPW = B // NW
C = 32
NCHUNK = BPW // C
GPC = C * D // 16

DENSE_ROWS = 64
SEQ_ROWS = [512, 512, 512, 64, 64]
SEQ_OFF = [N_DENSE * DENSE_ROWS + sum(SEQ_ROWS[:i]) for i in range(N_SEQ)]
SMALL_ROWS = N_DENSE * DENSE_ROWS + sum(SEQ_ROWS)
SPROWS0 = SMALL_ROWS
VAL_ROWS = SPROWS0 + N_SPARSE * 4 * C

_mesh = plsc.VectorSubcoreMesh(core_axis_name="c", subcore_axis_name="s")


def _body(*refs):
    w_refs = refs[:N_SPARSE]
    small_hbm = refs[N_SPARSE]
    idxm_hbm = refs[N_SPARSE + 1]
    out = refs[N_SPARSE + 2]
    (tbl_v, chunk0_v, chunk1_v, idx_v, glist_v,
     semA, semB, semO0, semO1) = refs[N_SPARSE + 3:]

    wid = lax.axis_index("c") * NS + lax.axis_index("s")
    base = wid * BPW

    pltpu.sync_copy(small_hbm, tbl_v.at[pl.ds(0, SMALL_ROWS), :])

    lane = lax.iota(jnp.int32, 16)
    lane4 = lane >> 2
    dvec = lane & 3

    def do_chunk(ci2, p, chunk_v, semO):
        ci = ci2 * 2 + p
        cb = base + ci * C

        @pl.when(ci2 > 0)
        def _():
            for d in range(D):
                pltpu.make_async_copy(
                    chunk_v.at[d],
                    out.at[:, 0, pl.ds(d * 128, C)], semO).wait()

        pltpu.sync_copy(idxm_hbm.at[:, pl.ds(cb, C)], idx_v)

        GQ = C // 4
        def gl_body(q, _):
            tvec = jnp.full((16,), 0, jnp.int32) + (q >> 3)
            cvec = (q & 7) * 4 + lane4
            iv = plsc.load_gather(idx_v, [tvec, cvec])
            r = ((iv >> 7) << 6) + (dvec << 4) + ((iv >> 3) & 15)
            glist_v[pl.ds(q * 16, 16)] = r
            return ()
        lax.fori_loop(0, N_SPARSE * GQ, gl_body, ())

        gps = []
        for t in range(N_SPARSE):
            gps.append(pltpu.async_copy(
                w_refs[t].at[glist_v.at[pl.ds(t * 4 * C, 4 * C)]],
                tbl_v.at[pl.ds(SPROWS0 + t * 4 * C, 4 * C), :], semB))

        for s in range(N_SEQ):
            A = SEQ_OFF[s]

            def k_body(k, _, s=s, A=A):
                jrow = jnp.full((16,), 0, jnp.int32) + (N_SINGLE + s * SEQ_LEN + k)
                oof = (N_SINGLE + s * SEQ_LEN + k) * D
                for g in range(GPC):
                    cvec = g * 4 + lane4
                    iv = plsc.load_gather(idx_v, [jrow, cvec])
                    rowv = A + ((iv >> 7) << 6) + (dvec << 4) + ((iv >> 3) & 15)
                    val = plsc.load_gather(tbl_v, [rowv, iv & 7])
                    plsc.store_scatter(chunk_v, [dvec, jrow, cvec], val)
                return ()
            lax.fori_loop(0, SEQ_LEN, k_body, ())

        for gp in gps:
            gp.wait()

        def t_body(t, _):
            flag = t < N_SPARSE
            rb = jnp.where(flag, SPROWS0 + t * 4 * C,
                           (t - N_SPARSE) * DENSE_ROWS)
            jrow = jnp.full((16,), 0, jnp.int32) + t
            for g in range(GPC):
                cvec = g * 4 + lane4
                iv = plsc.load_gather(idx_v, [jrow, cvec])
                rowv = jnp.where(
                    flag, rb + cvec * 4 + dvec,
                    rb + ((iv >> 7) << 6) + (dvec << 4) + ((iv >> 3) & 15))
                val = plsc.load_gather(tbl_v, [rowv, iv & 7])
                plsc.store_scatter(chunk_v, [dvec, jrow, cvec], val)
            return ()
        lax.fori_loop(0, N_SINGLE, t_body, ())

        q = cb >> 7
        o = cb & 127
        for d in range(D):
            off = pl.multiple_of(d * 128 + o, 32)
            pltpu.async_copy(chunk_v.at[d],
                             out.at[:, q, pl.ds(off, C)], semO)

    def chunk_body(ci2, _):
        do_chunk(ci2, 0, chunk0_v, semO0)
        do_chunk(ci2, 1, chunk1_v, semO1)
        return ()

    lax.fori_loop(0, NCHUNK // 2, chunk_body, ())

    for chunk_v, semO in ((chunk0_v, semO0), (chunk1_v, semO1)):
        for d in range(D):
            pltpu.make_async_copy(
                chunk_v.at[d], out.at[:, 0, pl.ds(d * 128, C)], semO).wait()


_call = functools.partial(
    pl.kernel,
    out_type=jax.ShapeDtypeStruct((NCOL, 128, 512), jnp.float32),
    mesh=_mesh,
    compiler_params=pltpu.CompilerParams(use_tc_tiling_on_sc=False,
                                         needs_layout_passes=False),
    scratch_types=[
        pltpu.VMEM((VAL_ROWS, 8), jnp.float32),
        pltpu.VMEM((D, NCOL, C), jnp.float32),
        pltpu.VMEM((D, NCOL, C), jnp.float32),
        pltpu.VMEM((NCOL, C), jnp.int32),
        pltpu.VMEM((N_SPARSE * 4 * C,), jnp.int32),
        pltpu.SemaphoreType.DMA,
        pltpu.SemaphoreType.DMA,
        pltpu.SemaphoreType.DMA,
        pltpu.SemaphoreType.DMA,
    ],
)(_body)


def kernel(sparse_0, W_sparse_0, sparse_1, W_sparse_1, sparse_2, W_sparse_2, sparse_3, W_sparse_3, sparse_4, W_sparse_4, sparse_5, W_sparse_5, sparse_6, W_sparse_6, sparse_7, W_sparse_7, sparse_8, W_sparse_8, sparse_9, W_sparse_9, sparse_10, W_sparse_10, sparse_11, W_sparse_11, sparse_12, W_sparse_12, sparse_13, W_sparse_13, sparse_14, W_sparse_14, sparse_15, W_sparse_15, sparse_16, W_sparse_16, sparse_17, W_sparse_17, sparse_18, W_sparse_18, sparse_19, W_sparse_19, sparse_20, W_sparse_20, sparse_21, W_sparse_21, sparse_22, W_sparse_22, sparse_23, W_sparse_23, sparse_24, W_sparse_24, sparse_25, W_sparse_25, dense_0, W_dense_0, dense_1, W_dense_1, dense_2, W_dense_2, dense_3, W_dense_3, dense_4, W_dense_4, dense_5, W_dense_5, dense_6, W_dense_6, dense_7, W_dense_7, dense_8, W_dense_8, dense_9, W_dense_9, dense_10, W_dense_10, dense_11, W_dense_11, dense_12, W_dense_12, register_game_seq, W_register_game_seq, active_game_seq, W_active_game_seq, pay_game_seq, W_pay_game_seq, onlinetime_seq, W_onlinetime_seq, payment_seq, W_payment_seq):
    kw = dict(locals())
    seq_names = ["register_game_seq", "active_game_seq", "pay_game_seq",
                 "onlinetime_seq", "payment_seq"]
    def _view8(w):
        v = w.shape[0]
        vp = -(-v // 128) * 128
        wp = jnp.pad(w, ((0, vp - v), (0, 0)))
        return wp.reshape(vp // 128, 128, 4).transpose(0, 2, 1).reshape(-1, 8)

    ws = [_view8(kw[f"W_sparse_{i}"]) for i in range(N_SPARSE)]
    small = jnp.concatenate(
        [_view8(kw[f"W_dense_{i}"]) for i in range(N_DENSE)]
        + [_view8(kw["W_" + n]) for n in seq_names], axis=0)
    idxm = jnp.concatenate(
        [jnp.stack([kw[f"sparse_{i}"] for i in range(N_SPARSE)]
                   + [kw[f"dense_{i}"] for i in range(N_DENSE)], axis=0)]
        + [kw[n].T for n in seq_names], axis=0)
    out3 = _call(*ws, small, idxm)
    return (out3.reshape(NCOL, 128, D, 128)
            .transpose(1, 3, 0, 2).reshape(B, NCOL, D))


# speedup vs baseline: 26.6428x; 1.0314x over previous
import functools

import jax
import jax.numpy as jnp
from jax import lax
from jax.experimental import pallas as pl
from jax.experimental.pallas import tpu as pltpu
from jax.experimental.pallas import tpu_sc as plsc

B = 16384
D = 4
N_SPARSE = 26
N_DENSE = 13
N_SEQ = 5
SEQ_LEN = 20
N_SINGLE = N_SPARSE + N_DENSE
NCOL = N_SINGLE + N_SEQ * SEQ_LEN
ROWF = NCOL * D
NC, NS = 2, 16
NW = NC * NS
BPW = B // NW
C = 32
NCHUNK = BPW // C
GPC = C * D // 16

DENSE_ROWS = 64
SEQ_ROWS = [512, 512, 512, 64, 64]
SEQ_OFF = [N_DENSE * DENSE_ROWS + sum(SEQ_ROWS[:i]) for i in range(N_SEQ)]
SMALL_ROWS = N_DENSE * DENSE_ROWS + sum(SEQ_ROWS)
SPROWS0 = SMALL_ROWS
VAL_ROWS = SPROWS0 + N_SPARSE * 4 * C

_mesh = plsc.VectorSubcoreMesh(core_axis_name="c", subcore_axis_name="s")


def _body(*refs):
    w_refs = refs[:N_SPARSE]
    small_hbm = refs[N_SPARSE]
    idxm_hbm = refs[N_SPARSE + 1]
    out = refs[N_SPARSE + 2]
    (tbl_v, chunk0_v, chunk1_v, idx0_v, idx1_v, glist_v,
     semA0, semA1, semB, semO0, semO1) = refs[N_SPARSE + 3:]

    wid = lax.axis_index("c") * NS + lax.axis_index("s")
    base = wid * BPW

    pltpu.sync_copy(small_hbm, tbl_v.at[pl.ds(0, SMALL_ROWS), :])

    lane = lax.iota(jnp.int32, 16)
    lane4 = lane >> 2
    dvec = lane & 3

    def stage_idx(ci, idx_b, semA):
        cb = base + ci * C
        pltpu.async_copy(idxm_hbm.at[:, pl.ds(cb, C)], idx_b, semA)

    def wait_idx(idx_b, semA):
        pltpu.make_async_copy(
            idxm_hbm.at[:, pl.ds(0, C)], idx_b, semA).wait()

    def build_glist(idx_b):
        def gl_body(q, _):
            tvec = jnp.full((16,), 0, jnp.int32) + (q >> 3)
            cvec = (q & 7) * 4 + lane4
            iv = plsc.load_gather(idx_b, [tvec, cvec])
            r = ((iv >> 7) << 6) + (dvec << 4) + ((iv >> 3) & 15)
            glist_v[pl.ds(q * 16, 16)] = r
            return ()
        lax.fori_loop(0, N_SPARSE * (C // 4), gl_body, ())

    def fire_gathers():
        for t in range(N_SPARSE):
            pltpu.async_copy(
                w_refs[t].at[glist_v.at[pl.ds(t * 4 * C, 4 * C)]],
                tbl_v.at[pl.ds(SPROWS0 + t * 4 * C, 4 * C), :], semB)

    def drain_gathers():
        for t in range(N_SPARSE):
            pltpu.make_async_copy(
                w_refs[t].at[glist_v.at[pl.ds(t * 4 * C, 4 * C)]],
                tbl_v.at[pl.ds(SPROWS0 + t * 4 * C, 4 * C), :], semB).wait()

    stage_idx(0, idx0_v, semA0)
    wait_idx(idx0_v, semA0)
    build_glist(idx0_v)
    fire_gathers()
    stage_idx(1, idx1_v, semA1)

    def do_chunk(ci2, p, chunk_v, semO, idx_b, semA_b, idx_n, semA_n):
        ci = ci2 * 2 + p
        cb = base + ci * C

        @pl.when(ci2 > 0)
        def _():
            for d in range(D):
                pltpu.make_async_copy(
                    chunk_v.at[d],
                    out.at[:, 0, pl.ds(d * 128, C)], semO).wait()

        for s in range(N_SEQ):
            A = SEQ_OFF[s]

            def k_body(k, _, s=s, A=A):
                jrow = jnp.full((16,), 0, jnp.int32) + (N_SINGLE + s * SEQ_LEN + k)
                for g in range(GPC):
                    cvec = g * 4 + lane4
                    iv = plsc.load_gather(idx_b, [jrow, cvec])
                    rowv = A + ((iv >> 7) << 6) + (dvec << 4) + ((iv >> 3) & 15)
                    val = plsc.load_gather(tbl_v, [rowv, iv & 7])
                    plsc.store_scatter(chunk_v, [dvec, jrow, cvec], val)
                return ()
            lax.fori_loop(0, SEQ_LEN, k_body, ())

        drain_gathers()

        def t_body(t, _):
            flag = t < N_SPARSE
            rb = jnp.where(flag, SPROWS0 + t * 4 * C,
                           (t - N_SPARSE) * DENSE_ROWS)
            jrow = jnp.full((16,), 0, jnp.int32) + t
            for g in range(GPC):
                cvec = g * 4 + lane4
                iv = plsc.load_gather(idx_b, [jrow, cvec])
                rowv = jnp.where(
                    flag, rb + cvec * 4 + dvec,
                    rb + ((iv >> 7) << 6) + (dvec << 4) + ((iv >> 3) & 15))
                val = plsc.load_gather(tbl_v, [rowv, iv & 7])
                plsc.store_scatter(chunk_v, [dvec, jrow, cvec], val)
            return ()
        lax.fori_loop(0, N_SINGLE, t_body, ())

        q = cb >> 7
        o = cb & 127
        for d in range(D):
            off = pl.multiple_of(d * 128 + o, 32)
            pltpu.async_copy(chunk_v.at[d],
                             out.at[:, q, pl.ds(off, C)], semO)

        @pl.when(ci < NCHUNK - 1)
        def _():
            wait_idx(idx_n, semA_n)
            build_glist(idx_n)
            fire_gathers()

            @pl.when(ci < NCHUNK - 2)
            def _():
                stage_idx(ci + 2, idx_b, semA_b)

    def chunk_body(ci2, _):
        do_chunk(ci2, 0, chunk0_v, semO0, idx0_v, semA0, idx1_v, semA1)
        do_chunk(ci2, 1, chunk1_v, semO1, idx1_v, semA1, idx0_v, semA0)
        return ()

    lax.fori_loop(0, NCHUNK // 2, chunk_body, ())

    for chunk_v, semO in ((chunk0_v, semO0), (chunk1_v, semO1)):
        for d in range(D):
            pltpu.make_async_copy(
                chunk_v.at[d], out.at[:, 0, pl.ds(d * 128, C)], semO).wait()


_call = functools.partial(
    pl.kernel,
    out_type=jax.ShapeDtypeStruct((NCOL, 128, 512), jnp.float32),
    mesh=_mesh,
    compiler_params=pltpu.CompilerParams(use_tc_tiling_on_sc=False,
                                         needs_layout_passes=False),
    scratch_types=[
        pltpu.VMEM((VAL_ROWS, 8), jnp.float32),
        pltpu.VMEM((D, NCOL, C), jnp.float32),
        pltpu.VMEM((D, NCOL, C), jnp.float32),
        pltpu.VMEM((NCOL, C), jnp.int32),
        pltpu.VMEM((NCOL, C), jnp.int32),
        pltpu.VMEM((N_SPARSE * 4 * C,), jnp.int32),
        pltpu.SemaphoreType.DMA,
        pltpu.SemaphoreType.DMA,
        pltpu.SemaphoreType.DMA,
        pltpu.SemaphoreType.DMA,
        pltpu.SemaphoreType.DMA,
    ],
)(_body)


def kernel(sparse_0, W_sparse_0, sparse_1, W_sparse_1, sparse_2, W_sparse_2, sparse_3, W_sparse_3, sparse_4, W_sparse_4, sparse_5, W_sparse_5, sparse_6, W_sparse_6, sparse_7, W_sparse_7, sparse_8, W_sparse_8, sparse_9, W_sparse_9, sparse_10, W_sparse_10, sparse_11, W_sparse_11, sparse_12, W_sparse_12, sparse_13, W_sparse_13, sparse_14, W_sparse_14, sparse_15, W_sparse_15, sparse_16, W_sparse_16, sparse_17, W_sparse_17, sparse_18, W_sparse_18, sparse_19, W_sparse_19, sparse_20, W_sparse_20, sparse_21, W_sparse_21, sparse_22, W_sparse_22, sparse_23, W_sparse_23, sparse_24, W_sparse_24, sparse_25, W_sparse_25, dense_0, W_dense_0, dense_1, W_dense_1, dense_2, W_dense_2, dense_3, W_dense_3, dense_4, W_dense_4, dense_5, W_dense_5, dense_6, W_dense_6, dense_7, W_dense_7, dense_8, W_dense_8, dense_9, W_dense_9, dense_10, W_dense_10, dense_11, W_dense_11, dense_12, W_dense_12, register_game_seq, W_register_game_seq, active_game_seq, W_active_game_seq, pay_game_seq, W_pay_game_seq, onlinetime_seq, W_onlinetime_seq, payment_seq, W_payment_seq):
    kw = dict(locals())
    seq_names = ["register_game_seq", "active_game_seq", "pay_game_seq",
                 "onlinetime_seq", "payment_seq"]
    def _view8(w):
        v = w.shape[0]
        vp = -(-v // 128) * 128
        wp = jnp.pad(w, ((0, vp - v), (0, 0)))
        return wp.reshape(vp // 128, 128, 4).transpose(0, 2, 1).reshape(-1, 8)

    ws = [_view8(kw[f"W_sparse_{i}"]) for i in range(N_SPARSE)]
    small = jnp.concatenate(
        [_view8(kw[f"W_dense_{i}"]) for i in range(N_DENSE)]
        + [_view8(kw["W_" + n]) for n in seq_names], axis=0)
    idxm = jnp.concatenate(
        [jnp.stack([kw[f"sparse_{i}"] for i in range(N_SPARSE)]
                   + [kw[f"dense_{i}"] for i in range(N_DENSE)], axis=0)]
        + [kw[n].T for n in seq_names], axis=0)
    out3 = _call(*ws, small, idxm)
    return (out3.reshape(NCOL, 128, D, 128)
            .transpose(1, 3, 0, 2).reshape(B, NCOL, D))


# speedup vs baseline: 45.2069x; 1.6968x over previous
import functools

import jax
import jax.numpy as jnp
from jax import lax
from jax.experimental import pallas as pl
from jax.experimental.pallas import tpu as pltpu
from jax.experimental.pallas import tpu_sc as plsc

B = 16384
D = 4
N_SPARSE = 26
N_DENSE = 13
N_SEQ = 5
SEQ_LEN = 20
N_SINGLE = N_SPARSE + N_DENSE
NCOL = N_SINGLE + N_SEQ * SEQ_LEN
ROWF = NCOL * D
NC, NS = 2, 16
NW = NC * NS
BPW = B // NW
C = 32
NCHUNK = BPW // C
GPC = C * D // 16

DENSE_ROWS = 64
SEQ_ROWS = [512, 512, 512, 64, 64]
SEQ_OFF = [N_DENSE * DENSE_ROWS + sum(SEQ_ROWS[:i]) for i in range(N_SEQ)]
SMALL_ROWS = N_DENSE * DENSE_ROWS + sum(SEQ_ROWS)
SPROWS0 = SMALL_ROWS
VAL_ROWS = SPROWS0 + N_SPARSE * 4 * C

_mesh = plsc.VectorSubcoreMesh(core_axis_name="c", subcore_axis_name="s")


def _body(*refs):
    w_refs = refs[:N_SPARSE]
    small_hbm = refs[N_SPARSE]
    idxm_hbm = refs[N_SPARSE + 1]
    out = refs[N_SPARSE + 2]
    (tbl_v, chunk0_v, chunk1_v, idx0_v, idx1_v, glist_v,
     semA0, semA1, semB, semO0, semO1) = refs[N_SPARSE + 3:]

    wid = lax.axis_index("c") * NS + lax.axis_index("s")
    base = wid * BPW

    pltpu.sync_copy(small_hbm, tbl_v.at[pl.ds(0, SMALL_ROWS), :])

    lane = lax.iota(jnp.int32, 16)
    lane4 = lane >> 2
    dvec = lane & 3

    def stage_idx(ci, idx_b, semA):
        cb = base + ci * C
        pltpu.async_copy(idxm_hbm.at[:, pl.ds(cb, C)], idx_b, semA)

    def wait_idx(idx_b, semA):
        pltpu.make_async_copy(
            idxm_hbm.at[:, pl.ds(0, C)], idx_b, semA).wait()

    def build_glist(idx_b):
        @plsc.parallel_loop(0, N_SPARSE * (C // 4), unroll=2)
        def gl_body(q):
            tvec = jnp.full((16,), 0, jnp.int32) + (q >> 3)
            cvec = (q & 7) * 4 + lane4
            iv = plsc.load_gather(idx_b, [tvec, cvec])
            r = ((iv >> 7) << 6) + (dvec << 4) + ((iv >> 3) & 15)
            glist_v[pl.ds(q * 16, 16)] = r

    def fire_gathers():
        for t in range(N_SPARSE):
            pltpu.async_copy(
                w_refs[t].at[glist_v.at[pl.ds(t * 4 * C, 4 * C)]],
                tbl_v.at[pl.ds(SPROWS0 + t * 4 * C, 4 * C), :], semB)

    def drain_gathers():
        for t in range(N_SPARSE):
            pltpu.make_async_copy(
                w_refs[t].at[glist_v.at[pl.ds(t * 4 * C, 4 * C)]],
                tbl_v.at[pl.ds(SPROWS0 + t * 4 * C, 4 * C), :], semB).wait()

    stage_idx(0, idx0_v, semA0)
    wait_idx(idx0_v, semA0)
    build_glist(idx0_v)
    fire_gathers()
    stage_idx(1, idx1_v, semA1)

    def do_chunk(ci2, p, chunk_v, semO, idx_b, semA_b, idx_n, semA_n):
        ci = ci2 * 2 + p
        cb = base + ci * C

        @pl.when(ci2 > 0)
        def _():
            for d in range(D):
                pltpu.make_async_copy(
                    chunk_v.at[d],
                    out.at[:, 0, pl.ds(d * 128, C)], semO).wait()

        for s in range(N_SEQ):
            A = SEQ_OFF[s]

            @plsc.parallel_loop(0, SEQ_LEN, unroll=1)
            def k_body(k, s=s, A=A):
                jrow = jnp.full((16,), 0, jnp.int32) + (N_SINGLE + s * SEQ_LEN + k)
                for g in range(GPC):
                    cvec = g * 4 + lane4
                    iv = plsc.load_gather(idx_b, [jrow, cvec])
                    rowv = A + ((iv >> 7) << 6) + (dvec << 4) + ((iv >> 3) & 15)
                    val = plsc.load_gather(tbl_v, [rowv, iv & 7])
                    plsc.store_scatter(chunk_v, [dvec, jrow, cvec], val)

        drain_gathers()

        @plsc.parallel_loop(0, N_SINGLE, unroll=1)
        def t_body(t):
            flag = t < N_SPARSE
            rb = jnp.where(flag, SPROWS0 + t * 4 * C,
                           (t - N_SPARSE) * DENSE_ROWS)
            jrow = jnp.full((16,), 0, jnp.int32) + t
            for g in range(GPC):
                cvec = g * 4 + lane4
                iv = plsc.load_gather(idx_b, [jrow, cvec])
                rowv = jnp.where(
                    flag, rb + cvec * 4 + dvec,
                    rb + ((iv >> 7) << 6) + (dvec << 4) + ((iv >> 3) & 15))
                val = plsc.load_gather(tbl_v, [rowv, iv & 7])
                plsc.store_scatter(chunk_v, [dvec, jrow, cvec], val)

        q = cb >> 7
        o = cb & 127
        for d in range(D):
            off = pl.multiple_of(d * 128 + o, 32)
            pltpu.async_copy(chunk_v.at[d],
                             out.at[:, q, pl.ds(off, C)], semO)

        @pl.when(ci < NCHUNK - 1)
        def _():
            wait_idx(idx_n, semA_n)
            build_glist(idx_n)
            fire_gathers()

            @pl.when(ci < NCHUNK - 2)
            def _():
                stage_idx(ci + 2, idx_b, semA_b)

    def chunk_body(ci2, _):
        do_chunk(ci2, 0, chunk0_v, semO0, idx0_v, semA0, idx1_v, semA1)
        do_chunk(ci2, 1, chunk1_v, semO1, idx1_v, semA1, idx0_v, semA0)
        return ()

    lax.fori_loop(0, NCHUNK // 2, chunk_body, ())

    for chunk_v, semO in ((chunk0_v, semO0), (chunk1_v, semO1)):
        for d in range(D):
            pltpu.make_async_copy(
                chunk_v.at[d], out.at[:, 0, pl.ds(d * 128, C)], semO).wait()


_call = functools.partial(
    pl.kernel,
    out_type=jax.ShapeDtypeStruct((NCOL, 128, 512), jnp.float32),
    mesh=_mesh,
    compiler_params=pltpu.CompilerParams(use_tc_tiling_on_sc=False,
                                         needs_layout_passes=False),
    scratch_types=[
        pltpu.VMEM((VAL_ROWS, 8), jnp.float32),
        pltpu.VMEM((D, NCOL, C), jnp.float32),
        pltpu.VMEM((D, NCOL, C), jnp.float32),
        pltpu.VMEM((NCOL, C), jnp.int32),
        pltpu.VMEM((NCOL, C), jnp.int32),
        pltpu.VMEM((N_SPARSE * 4 * C,), jnp.int32),
        pltpu.SemaphoreType.DMA,
        pltpu.SemaphoreType.DMA,
        pltpu.SemaphoreType.DMA,
        pltpu.SemaphoreType.DMA,
        pltpu.SemaphoreType.DMA,
    ],
)(_body)


def kernel(sparse_0, W_sparse_0, sparse_1, W_sparse_1, sparse_2, W_sparse_2, sparse_3, W_sparse_3, sparse_4, W_sparse_4, sparse_5, W_sparse_5, sparse_6, W_sparse_6, sparse_7, W_sparse_7, sparse_8, W_sparse_8, sparse_9, W_sparse_9, sparse_10, W_sparse_10, sparse_11, W_sparse_11, sparse_12, W_sparse_12, sparse_13, W_sparse_13, sparse_14, W_sparse_14, sparse_15, W_sparse_15, sparse_16, W_sparse_16, sparse_17, W_sparse_17, sparse_18, W_sparse_18, sparse_19, W_sparse_19, sparse_20, W_sparse_20, sparse_21, W_sparse_21, sparse_22, W_sparse_22, sparse_23, W_sparse_23, sparse_24, W_sparse_24, sparse_25, W_sparse_25, dense_0, W_dense_0, dense_1, W_dense_1, dense_2, W_dense_2, dense_3, W_dense_3, dense_4, W_dense_4, dense_5, W_dense_5, dense_6, W_dense_6, dense_7, W_dense_7, dense_8, W_dense_8, dense_9, W_dense_9, dense_10, W_dense_10, dense_11, W_dense_11, dense_12, W_dense_12, register_game_seq, W_register_game_seq, active_game_seq, W_active_game_seq, pay_game_seq, W_pay_game_seq, onlinetime_seq, W_onlinetime_seq, payment_seq, W_payment_seq):
    kw = dict(locals())
    seq_names = ["register_game_seq", "active_game_seq", "pay_game_seq",
                 "onlinetime_seq", "payment_seq"]
    def _view8(w):
        v = w.shape[0]
        vp = -(-v // 128) * 128
        wp = jnp.pad(w, ((0, vp - v), (0, 0)))
        return wp.reshape(vp // 128, 128, 4).transpose(0, 2, 1).reshape(-1, 8)

    ws = [_view8(kw[f"W_sparse_{i}"]) for i in range(N_SPARSE)]
    small = jnp.concatenate(
        [_view8(kw[f"W_dense_{i}"]) for i in range(N_DENSE)]
        + [_view8(kw["W_" + n]) for n in seq_names], axis=0)
    idxm = jnp.concatenate(
        [jnp.stack([kw[f"sparse_{i}"] for i in range(N_SPARSE)]
                   + [kw[f"dense_{i}"] for i in range(N_DENSE)], axis=0)]
        + [kw[n].T for n in seq_names], axis=0)
    out3 = _call(*ws, small, idxm)
    return (out3.reshape(NCOL, 128, D, 128)
            .transpose(1, 3, 0, 2).reshape(B, NCOL, D))


# speedup vs baseline: 45.2951x; 1.0020x over previous
import functools

import jax
import jax.numpy as jnp
from jax import lax
from jax.experimental import pallas as pl
from jax.experimental.pallas import tpu as pltpu
from jax.experimental.pallas import tpu_sc as plsc

B = 16384
D = 4
N_SPARSE = 26
N_DENSE = 13
N_SEQ = 5
SEQ_LEN = 20
N_SINGLE = N_SPARSE + N_DENSE
NCOL = N_SINGLE + N_SEQ * SEQ_LEN
ROWF = NCOL * D
NC, NS = 2, 16
NW = NC * NS
BPW = B // NW
C = 32
NCHUNK = BPW // C
GPC = C * D // 16

DENSE_ROWS = 64
SEQ_ROWS = [512, 512, 512, 64, 64]
SEQ_OFF = [N_DENSE * DENSE_ROWS + sum(SEQ_ROWS[:i]) for i in range(N_SEQ)]
SMALL_ROWS = N_DENSE * DENSE_ROWS + sum(SEQ_ROWS)
SPROWS0 = SMALL_ROWS
VAL_ROWS = SPROWS0 + N_SPARSE * 4 * C

_mesh = plsc.VectorSubcoreMesh(core_axis_name="c", subcore_axis_name="s")


def _body(*refs):
    w_refs = refs[:N_SPARSE]
    small_hbm = refs[N_SPARSE]
    idxm_hbm = refs[N_SPARSE + 1]
    out = refs[N_SPARSE + 2]
    (tbl_v, chunk0_v, chunk1_v, idx0_v, idx1_v, glist_v,
     semA0, semA1, semB, semO0, semO1) = refs[N_SPARSE + 3:]

    wid = lax.axis_index("c") * NS + lax.axis_index("s")
    base = wid * BPW

    pltpu.sync_copy(small_hbm, tbl_v.at[pl.ds(0, SMALL_ROWS), :])

    lane = lax.iota(jnp.int32, 16)
    lane4 = lane >> 2
    dvec = lane & 3

    def stage_idx(ci, idx_b, semA):
        cb = base + ci * C
        pltpu.async_copy(idxm_hbm.at[:, pl.ds(cb, C)], idx_b, semA)

    def wait_idx(idx_b, semA):
        pltpu.make_async_copy(
            idxm_hbm.at[:, pl.ds(0, C)], idx_b, semA).wait()

    def build_glist(idx_b):
        @plsc.parallel_loop(0, N_SPARSE * (C // 4), unroll=2)
        def gl_body(q):
            tvec = jnp.full((16,), 0, jnp.int32) + (q >> 3)
            cvec = (q & 7) * 4 + lane4
            iv = plsc.load_gather(idx_b, [tvec, cvec])
            r = ((iv >> 7) << 6) + (dvec << 4) + ((iv >> 3) & 15)
            glist_v[pl.ds(q * 16, 16)] = r

    def fire_gathers():
        for t in range(N_SPARSE):
            pltpu.async_copy(
                w_refs[t].at[glist_v.at[pl.ds(t * 4 * C, 4 * C)]],
                tbl_v.at[pl.ds(SPROWS0 + t * 4 * C, 4 * C), :], semB)

    def drain_gathers():
        for t in range(N_SPARSE):
            pltpu.make_async_copy(
                w_refs[t].at[glist_v.at[pl.ds(t * 4 * C, 4 * C)]],
                tbl_v.at[pl.ds(SPROWS0 + t * 4 * C, 4 * C), :], semB).wait()

    stage_idx(0, idx0_v, semA0)
    wait_idx(idx0_v, semA0)
    build_glist(idx0_v)
    fire_gathers()
    stage_idx(1, idx1_v, semA1)

    def do_chunk(ci2, p, chunk_v, semO, idx_b, semA_b, idx_n, semA_n):
        ci = ci2 * 2 + p
        cb = base + ci * C

        @pl.when(ci2 > 0)
        def _():
            for d in range(D):
                pltpu.make_async_copy(
                    chunk_v.at[d],
                    out.at[:, 0, pl.ds(d * 128, C)], semO).wait()

        @plsc.parallel_loop(0, N_SEQ * SEQ_LEN, unroll=2)
        def k_body(jq):
            A = (SEQ_OFF[0]
                 + jnp.where(jq >= 20, SEQ_ROWS[0], 0)
                 + jnp.where(jq >= 40, SEQ_ROWS[1], 0)
                 + jnp.where(jq >= 60, SEQ_ROWS[2], 0)
                 + jnp.where(jq >= 80, SEQ_ROWS[3], 0))
            jrow = jnp.full((16,), 0, jnp.int32) + (N_SINGLE + jq)
            for g in range(GPC):
                cvec = g * 4 + lane4
                iv = plsc.load_gather(idx_b, [jrow, cvec])
                rowv = A + ((iv >> 7) << 6) + (dvec << 4) + ((iv >> 3) & 15)
                val = plsc.load_gather(tbl_v, [rowv, iv & 7])
                plsc.store_scatter(chunk_v, [dvec, jrow, cvec], val)

        drain_gathers()

        @plsc.parallel_loop(0, N_SINGLE, unroll=1)
        def t_body(t):
            flag = t < N_SPARSE
            rb = jnp.where(flag, SPROWS0 + t * 4 * C,
                           (t - N_SPARSE) * DENSE_ROWS)
            jrow = jnp.full((16,), 0, jnp.int32) + t
            for g in range(GPC):
                cvec = g * 4 + lane4
                iv = plsc.load_gather(idx_b, [jrow, cvec])
                rowv = jnp.where(
                    flag, rb + cvec * 4 + dvec,
                    rb + ((iv >> 7) << 6) + (dvec << 4) + ((iv >> 3) & 15))
                val = plsc.load_gather(tbl_v, [rowv, iv & 7])
                plsc.store_scatter(chunk_v, [dvec, jrow, cvec], val)

        q = cb >> 7
        o = cb & 127
        for d in range(D):
            off = pl.multiple_of(d * 128 + o, 32)
            pltpu.async_copy(chunk_v.at[d],
                             out.at[:, q, pl.ds(off, C)], semO)

        @pl.when(ci < NCHUNK - 1)
        def _():
            wait_idx(idx_n, semA_n)
            build_glist(idx_n)
            fire_gathers()

            @pl.when(ci < NCHUNK - 2)
            def _():
                stage_idx(ci + 2, idx_b, semA_b)

    def chunk_body(ci2, _):
        do_chunk(ci2, 0, chunk0_v, semO0, idx0_v, semA0, idx1_v, semA1)
        do_chunk(ci2, 1, chunk1_v, semO1, idx1_v, semA1, idx0_v, semA0)
        return ()

    lax.fori_loop(0, NCHUNK // 2, chunk_body, ())

    for chunk_v, semO in ((chunk0_v, semO0), (chunk1_v, semO1)):
        for d in range(D):
            pltpu.make_async_copy(
                chunk_v.at[d], out.at[:, 0, pl.ds(d * 128, C)], semO).wait()


_call = functools.partial(
    pl.kernel,
    out_type=jax.ShapeDtypeStruct((NCOL, 128, 512), jnp.float32),
    mesh=_mesh,
    compiler_params=pltpu.CompilerParams(use_tc_tiling_on_sc=False,
                                         needs_layout_passes=False),
    scratch_types=[
        pltpu.VMEM((VAL_ROWS, 8), jnp.float32),
        pltpu.VMEM((D, NCOL, C), jnp.float32),
        pltpu.VMEM((D, NCOL, C), jnp.float32),
        pltpu.VMEM((NCOL, C), jnp.int32),
        pltpu.VMEM((NCOL, C), jnp.int32),
        pltpu.VMEM((N_SPARSE * 4 * C,), jnp.int32),
        pltpu.SemaphoreType.DMA,
        pltpu.SemaphoreType.DMA,
        pltpu.SemaphoreType.DMA,
        pltpu.SemaphoreType.DMA,
        pltpu.SemaphoreType.DMA,
    ],
)(_body)


def kernel(sparse_0, W_sparse_0, sparse_1, W_sparse_1, sparse_2, W_sparse_2, sparse_3, W_sparse_3, sparse_4, W_sparse_4, sparse_5, W_sparse_5, sparse_6, W_sparse_6, sparse_7, W_sparse_7, sparse_8, W_sparse_8, sparse_9, W_sparse_9, sparse_10, W_sparse_10, sparse_11, W_sparse_11, sparse_12, W_sparse_12, sparse_13, W_sparse_13, sparse_14, W_sparse_14, sparse_15, W_sparse_15, sparse_16, W_sparse_16, sparse_17, W_sparse_17, sparse_18, W_sparse_18, sparse_19, W_sparse_19, sparse_20, W_sparse_20, sparse_21, W_sparse_21, sparse_22, W_sparse_22, sparse_23, W_sparse_23, sparse_24, W_sparse_24, sparse_25, W_sparse_25, dense_0, W_dense_0, dense_1, W_dense_1, dense_2, W_dense_2, dense_3, W_dense_3, dense_4, W_dense_4, dense_5, W_dense_5, dense_6, W_dense_6, dense_7, W_dense_7, dense_8, W_dense_8, dense_9, W_dense_9, dense_10, W_dense_10, dense_11, W_dense_11, dense_12, W_dense_12, register_game_seq, W_register_game_seq, active_game_seq, W_active_game_seq, pay_game_seq, W_pay_game_seq, onlinetime_seq, W_onlinetime_seq, payment_seq, W_payment_seq):
    kw = dict(locals())
    seq_names = ["register_game_seq", "active_game_seq", "pay_game_seq",
                 "onlinetime_seq", "payment_seq"]
    def _view8(w):
        v = w.shape[0]
        vp = -(-v // 128) * 128
        wp = jnp.pad(w, ((0, vp - v), (0, 0)))
        return wp.reshape(vp // 128, 128, 4).transpose(0, 2, 1).reshape(-1, 8)

    ws = [_view8(kw[f"W_sparse_{i}"]) for i in range(N_SPARSE)]
    small = jnp.concatenate(
        [_view8(kw[f"W_dense_{i}"]) for i in range(N_DENSE)]
        + [_view8(kw["W_" + n]) for n in seq_names], axis=0)
    idxm = jnp.concatenate(
        [jnp.stack([kw[f"sparse_{i}"] for i in range(N_SPARSE)]
                   + [kw[f"dense_{i}"] for i in range(N_DENSE)], axis=0)]
        + [kw[n].T for n in seq_names], axis=0)
    out3 = _call(*ws, small, idxm)
    return (out3.reshape(NCOL, 128, D, 128)
            .transpose(1, 3, 0, 2).reshape(B, NCOL, D))
